# baseline (speedup 1.0000x reference)
import numpy as np

import jax
import jax.numpy as jnp
from jax.experimental import pallas as pl
from jax.experimental.pallas import tpu as pltpu

_N = 10000
_BM = 1000
_BK = 1024
_RB = _N // _BM
_CB = -(-_N // _BK)
_NPAD = _CB * _BK
_F1 = 128
_F2 = 64


def _dual(r, c):
    return _BK * (c + 1) <= _BM * r


def _make_schedule():
    rs, cs, ph = [], [], []
    for r in range(_RB):
        for c in range(_CB):
            rs.append(r)
            cs.append(c)
            ph.append(0)
    for r in range(_RB):
        for c in range(_CB):
            if not _dual(r, c):
                rs.append(r)
                cs.append(c)
                ph.append(1)
    return (np.asarray(rs, np.int32), np.asarray(cs, np.int32),
            np.asarray(ph, np.int32))


_RTAB, _CTAB, _PTAB = _make_schedule()
_NSTEPS = _RTAB.shape[0]


def _mm_kernel(a_ref, w_ref, o_ref):
    a = a_ref[...].astype(jnp.bfloat16)
    w = w_ref[...].astype(jnp.bfloat16)
    o_ref[...] = jnp.dot(a, w, preferred_element_type=jnp.float32)


def _lsm_kernel(h_ref, o_ref):
    h = h_ref[...]
    m = jnp.max(h, axis=0, keepdims=True)
    lse = jnp.log(jnp.sum(jnp.exp(h - m), axis=0, keepdims=True)) + m
    o_ref[...] = h - lse


def _fused_kernel(rtab_ref, ctab_ref, ptab_ref, adj_ref, s1_ref, b0_ref,
                  w1_ref, out_ref, h1p_ref, s_ref):
    t = pl.program_id(0)
    r = rtab_ref[t]
    c = ctab_ref[t]
    ph = ptab_ref[t]

    @pl.when(t == 0)
    def _init_s():
        s_ref[:, :_F1] = s1_ref[...]
        s_ref[:, _F1:] = jnp.zeros((_NPAD, _F2), jnp.float32)

    res = jnp.zeros((_BM, _F1 + _F2), jnp.float32)
    for k in range(_BK // 256):
        tk = adj_ref[:, 256 * k:256 * (k + 1)].astype(jnp.bfloat16)
        sk = s_ref[pl.ds(c * _BK + 256 * k, 256), :].astype(jnp.bfloat16)
        res = res + jnp.dot(tk, sk, preferred_element_type=jnp.float32)

    @pl.when(ph == 0)
    def _pass1():
        part = res[:, :_F1]

        @pl.when(c == 0)
        def _():
            h1p_ref[...] = part

        @pl.when(c != 0)
        def _():
            h1p_ref[...] = h1p_ref[...] + part

        @pl.when(c == _CB - 1)
        def _finalize_stripe():
            h1 = jnp.maximum(h1p_ref[...] + b0_ref[...], 0.0)
            s2_blk = jnp.dot(h1.astype(jnp.bfloat16),
                             w1_ref[...].astype(jnp.bfloat16),
                             preferred_element_type=jnp.float32)
            s_ref[pl.ds(r * _BM, _BM), _F1:] = s2_blk

    @pl.when(jnp.logical_or(ph == 1, _BK * (c + 1) <= _BM * r))
    def _pass2():
        contrib = res[:, _F1:]

        @pl.when(c == 0)
        def _():
            out_ref[pl.ds(r * _BM, _BM), :] = contrib

        @pl.when(c != 0)
        def _():
            out_ref[pl.ds(r * _BM, _BM), :] = (
                out_ref[pl.ds(r * _BM, _BM), :] + contrib)


def kernel(x, adj, W0, b0, W1, b1):
    x2d = x.reshape(_N, x.shape[-1])
    x_pad = jnp.pad(x2d, ((0, _NPAD - _N), (0, 0)))
    s1 = pl.pallas_call(
        _mm_kernel,
        out_shape=jax.ShapeDtypeStruct((_NPAD, _F1), jnp.float32),
    )(x_pad, W0)

    h2 = pl.pallas_call(
        _fused_kernel,
        grid_spec=pltpu.PrefetchScalarGridSpec(
            num_scalar_prefetch=3,
            grid=(_NSTEPS,),
            in_specs=[
                pl.BlockSpec((_BM, _BK),
                             lambda t, rt, ct, pt: (rt[t], ct[t])),
                pl.BlockSpec((_NPAD, _F1), lambda t, rt, ct, pt: (0, 0)),
                pl.BlockSpec((1, _F1), lambda t, rt, ct, pt: (0, 0)),
                pl.BlockSpec((_F1, _F2), lambda t, rt, ct, pt: (0, 0)),
            ],
            out_specs=pl.BlockSpec((_N, _F2), lambda t, rt, ct, pt: (0, 0)),
            scratch_shapes=[
                pltpu.VMEM((_BM, _F1), jnp.float32),
                pltpu.VMEM((_NPAD, _F1 + _F2), jnp.float32),
            ],
        ),
        out_shape=jax.ShapeDtypeStruct((_N, _F2), jnp.float32),
        compiler_params=pltpu.CompilerParams(
            dimension_semantics=("arbitrary",)),
    )(jnp.asarray(_RTAB), jnp.asarray(_CTAB), jnp.asarray(_PTAB),
      adj, s1, b0.reshape(1, -1), W1)

    out = pl.pallas_call(
        _lsm_kernel,
        out_shape=jax.ShapeDtypeStruct((_N, _F2), jnp.float32),
    )(h2)
    return out.reshape(1, _N, _F2)


# speedup vs baseline: 1.0063x; 1.0063x over previous
import numpy as np

import jax
import jax.numpy as jnp
from jax.experimental import pallas as pl
from jax.experimental.pallas import tpu as pltpu

_N = 10000
_BM = 1000
_BK = 1024
_RB = _N // _BM
_CB = -(-_N // _BK)
_NPAD = _CB * _BK
_F1 = 128
_F2 = 64


def _dual(r, c):
    return _BK * (c + 1) <= _BM * r


def _make_schedule():
    rs, cs, ph = [], [], []
    for r in range(_RB):
        for c in range(_CB):
            rs.append(r)
            cs.append(c)
            ph.append(0)
    for r in range(_RB):
        for c in range(_CB):
            if not _dual(r, c):
                rs.append(r)
                cs.append(c)
                ph.append(1)
    return (np.asarray(rs, np.int32), np.asarray(cs, np.int32),
            np.asarray(ph, np.int32))


_RTAB, _CTAB, _PTAB = _make_schedule()
_NSTEPS = _RTAB.shape[0]


def _mm_kernel(a_ref, w_ref, o_ref):
    a = a_ref[...].astype(jnp.bfloat16)
    w = w_ref[...].astype(jnp.bfloat16)
    o_ref[...] = jnp.dot(a, w, preferred_element_type=jnp.float32)


def _lsm_kernel(h_ref, o_ref):
    h = h_ref[...]
    m = jnp.max(h, axis=0, keepdims=True)
    lse = jnp.log(jnp.sum(jnp.exp(h - m), axis=0, keepdims=True)) + m
    o_ref[...] = h - lse


def _fused_kernel(rtab_ref, ctab_ref, ptab_ref, adj_ref, s1_ref, b0_ref,
                  w1_ref, out_ref, h1p_ref, s_ref):
    t = pl.program_id(0)
    r = rtab_ref[t]
    c = ctab_ref[t]
    ph = ptab_ref[t]

    @pl.when(t == 0)
    def _init_s():
        s_ref[:, :_F1] = s1_ref[...]
        s_ref[:, _F1:] = jnp.zeros((_NPAD, _F2), jnp.float32)

    res = jax.lax.dot_general(
        adj_ref[...], s_ref[pl.ds(c * _BK, _BK), :],
        (((1,), (0,)), ((), ())),
        precision=jax.lax.Precision.DEFAULT,
        preferred_element_type=jnp.float32)

    @pl.when(ph == 0)
    def _pass1():
        part = res[:, :_F1]

        @pl.when(c == 0)
        def _():
            h1p_ref[...] = part

        @pl.when(c != 0)
        def _():
            h1p_ref[...] = h1p_ref[...] + part

        @pl.when(c == _CB - 1)
        def _finalize_stripe():
            h1 = jnp.maximum(h1p_ref[...] + b0_ref[...], 0.0)
            s2_blk = jnp.dot(h1.astype(jnp.bfloat16),
                             w1_ref[...].astype(jnp.bfloat16),
                             preferred_element_type=jnp.float32)
            s_ref[pl.ds(r * _BM, _BM), _F1:] = s2_blk

    @pl.when(jnp.logical_or(ph == 1, _BK * (c + 1) <= _BM * r))
    def _pass2():
        contrib = res[:, _F1:]

        @pl.when(c == 0)
        def _():
            out_ref[pl.ds(r * _BM, _BM), :] = contrib

        @pl.when(c != 0)
        def _():
            out_ref[pl.ds(r * _BM, _BM), :] = (
                out_ref[pl.ds(r * _BM, _BM), :] + contrib)


def kernel(x, adj, W0, b0, W1, b1):
    x2d = x.reshape(_N, x.shape[-1])
    x_pad = jnp.pad(x2d, ((0, _NPAD - _N), (0, 0)))
    s1 = pl.pallas_call(
        _mm_kernel,
        out_shape=jax.ShapeDtypeStruct((_NPAD, _F1), jnp.float32),
    )(x_pad, W0)

    h2 = pl.pallas_call(
        _fused_kernel,
        grid_spec=pltpu.PrefetchScalarGridSpec(
            num_scalar_prefetch=3,
            grid=(_NSTEPS,),
            in_specs=[
                pl.BlockSpec((_BM, _BK),
                             lambda t, rt, ct, pt: (rt[t], ct[t])),
                pl.BlockSpec((_NPAD, _F1), lambda t, rt, ct, pt: (0, 0)),
                pl.BlockSpec((1, _F1), lambda t, rt, ct, pt: (0, 0)),
                pl.BlockSpec((_F1, _F2), lambda t, rt, ct, pt: (0, 0)),
            ],
            out_specs=pl.BlockSpec((_N, _F2), lambda t, rt, ct, pt: (0, 0)),
            scratch_shapes=[
                pltpu.VMEM((_BM, _F1), jnp.float32),
                pltpu.VMEM((_NPAD, _F1 + _F2), jnp.float32),
            ],
        ),
        out_shape=jax.ShapeDtypeStruct((_N, _F2), jnp.float32),
        compiler_params=pltpu.CompilerParams(
            dimension_semantics=("arbitrary",)),
    )(jnp.asarray(_RTAB), jnp.asarray(_CTAB), jnp.asarray(_PTAB),
      adj, s1, b0.reshape(1, -1), W1)

    out = pl.pallas_call(
        _lsm_kernel,
        out_shape=jax.ShapeDtypeStruct((_N, _F2), jnp.float32),
    )(h2)
    return out.reshape(1, _N, _F2)


# speedup vs baseline: 1.2230x; 1.2153x over previous
import numpy as np

import jax
import jax.numpy as jnp
from jax.experimental import pallas as pl
from jax.experimental.pallas import tpu as pltpu

_N = 10000
_BM = 1000
_BK = 2048
_RB = _N // _BM
_CB = -(-_N // _BK)
_NPAD = _CB * _BK
_F1 = 128
_F2 = 64


def _dual(r, c):
    return _BK * (c + 1) <= _BM * r


def _make_schedule():
    rs, cs, ph = [], [], []
    for r in range(_RB):
        for c in range(_CB):
            rs.append(r)
            cs.append(c)
            ph.append(0)
    for r in range(_RB):
        for c in range(_CB):
            if not _dual(r, c):
                rs.append(r)
                cs.append(c)
                ph.append(1)
    return (np.asarray(rs, np.int32), np.asarray(cs, np.int32),
            np.asarray(ph, np.int32))


_RTAB, _CTAB, _PTAB = _make_schedule()
_NSTEPS = _RTAB.shape[0]


def _mm_kernel(a_ref, w_ref, o_ref):
    a = a_ref[...].astype(jnp.bfloat16)
    w = w_ref[...].astype(jnp.bfloat16)
    o_ref[...] = jnp.dot(a, w, preferred_element_type=jnp.float32)


def _lsm_kernel(h_ref, o_ref):
    h = h_ref[...]
    m = jnp.max(h, axis=0, keepdims=True)
    lse = jnp.log(jnp.sum(jnp.exp(h - m), axis=0, keepdims=True)) + m
    o_ref[...] = h - lse


def _fused_kernel(rtab_ref, ctab_ref, ptab_ref, adj_ref, s1_ref, b0_ref,
                  w1_ref, out_ref, h1p_ref, s_ref):
    t = pl.program_id(0)
    r = rtab_ref[t]
    c = ctab_ref[t]
    ph = ptab_ref[t]

    @pl.when(t == 0)
    def _init_s():
        s_ref[:, :_F1] = s1_ref[...]
        s_ref[:, _F1:] = jnp.zeros((_NPAD, _F2), jnp.float32)

    res = jax.lax.dot_general(
        adj_ref[...], s_ref[pl.ds(c * _BK, _BK), :],
        (((1,), (0,)), ((), ())),
        precision=jax.lax.Precision.DEFAULT,
        preferred_element_type=jnp.float32)

    @pl.when(ph == 0)
    def _pass1():
        part = res[:, :_F1]

        @pl.when(c == 0)
        def _():
            h1p_ref[...] = part

        @pl.when(c != 0)
        def _():
            h1p_ref[...] = h1p_ref[...] + part

        @pl.when(c == _CB - 1)
        def _finalize_stripe():
            h1 = jnp.maximum(h1p_ref[...] + b0_ref[...], 0.0)
            s2_blk = jnp.dot(h1.astype(jnp.bfloat16),
                             w1_ref[...].astype(jnp.bfloat16),
                             preferred_element_type=jnp.float32)
            s_ref[pl.ds(r * _BM, _BM), _F1:] = s2_blk

    @pl.when(jnp.logical_or(ph == 1, _BK * (c + 1) <= _BM * r))
    def _pass2():
        contrib = res[:, _F1:]

        @pl.when(c == 0)
        def _():
            out_ref[pl.ds(r * _BM, _BM), :] = contrib

        @pl.when(c != 0)
        def _():
            out_ref[pl.ds(r * _BM, _BM), :] = (
                out_ref[pl.ds(r * _BM, _BM), :] + contrib)


def kernel(x, adj, W0, b0, W1, b1):
    x2d = x.reshape(_N, x.shape[-1])
    x_pad = jnp.pad(x2d, ((0, _NPAD - _N), (0, 0)))
    s1 = pl.pallas_call(
        _mm_kernel,
        out_shape=jax.ShapeDtypeStruct((_NPAD, _F1), jnp.float32),
    )(x_pad, W0)

    h2 = pl.pallas_call(
        _fused_kernel,
        grid_spec=pltpu.PrefetchScalarGridSpec(
            num_scalar_prefetch=3,
            grid=(_NSTEPS,),
            in_specs=[
                pl.BlockSpec((_BM, _BK),
                             lambda t, rt, ct, pt: (rt[t], ct[t])),
                pl.BlockSpec((_NPAD, _F1), lambda t, rt, ct, pt: (0, 0)),
                pl.BlockSpec((1, _F1), lambda t, rt, ct, pt: (0, 0)),
                pl.BlockSpec((_F1, _F2), lambda t, rt, ct, pt: (0, 0)),
            ],
            out_specs=pl.BlockSpec((_N, _F2), lambda t, rt, ct, pt: (0, 0)),
            scratch_shapes=[
                pltpu.VMEM((_BM, _F1), jnp.float32),
                pltpu.VMEM((_NPAD, _F1 + _F2), jnp.float32),
            ],
        ),
        out_shape=jax.ShapeDtypeStruct((_N, _F2), jnp.float32),
        compiler_params=pltpu.CompilerParams(
            dimension_semantics=("arbitrary",)),
    )(jnp.asarray(_RTAB), jnp.asarray(_CTAB), jnp.asarray(_PTAB),
      adj, s1, b0.reshape(1, -1), W1)

    out = pl.pallas_call(
        _lsm_kernel,
        out_shape=jax.ShapeDtypeStruct((_N, _F2), jnp.float32),
    )(h2)
    return out.reshape(1, _N, _F2)


# speedup vs baseline: 1.2669x; 1.0359x over previous
import numpy as np

import jax
import jax.numpy as jnp
from jax.experimental import pallas as pl
from jax.experimental.pallas import tpu as pltpu

_N = 10000
_BM = 1000
_BK = 2560
_RB = _N // _BM
_CB = -(-_N // _BK)
_NPAD = _CB * _BK
_F1 = 128
_F2 = 64


def _dual(r, c):
    return _BK * (c + 1) <= _BM * r


def _make_schedule():
    rs, cs, ph = [], [], []
    for r in range(_RB):
        for c in range(_CB):
            rs.append(r)
            cs.append(c)
            ph.append(0)
    for r in range(_RB):
        for c in range(_CB):
            if not _dual(r, c):
                rs.append(r)
                cs.append(c)
                ph.append(1)
    return (np.asarray(rs, np.int32), np.asarray(cs, np.int32),
            np.asarray(ph, np.int32))


_RTAB, _CTAB, _PTAB = _make_schedule()
_NSTEPS = _RTAB.shape[0]


def _mm_kernel(a_ref, w_ref, o_ref):
    a = a_ref[...].astype(jnp.bfloat16)
    w = w_ref[...].astype(jnp.bfloat16)
    o_ref[...] = jnp.dot(a, w, preferred_element_type=jnp.float32)


def _lsm_kernel(h_ref, o_ref):
    h = h_ref[...]
    m = jnp.max(h, axis=0, keepdims=True)
    lse = jnp.log(jnp.sum(jnp.exp(h - m), axis=0, keepdims=True)) + m
    o_ref[...] = h - lse


def _fused_kernel(rtab_ref, ctab_ref, ptab_ref, adj_ref, s1_ref, b0_ref,
                  w1_ref, out_ref, h1p_ref, s_ref):
    t = pl.program_id(0)
    r = rtab_ref[t]
    c = ctab_ref[t]
    ph = ptab_ref[t]

    @pl.when(t == 0)
    def _init_s():
        s_ref[:, :_F1] = s1_ref[...]
        s_ref[:, _F1:] = jnp.zeros((_NPAD, _F2), jnp.float32)

    res = jax.lax.dot_general(
        adj_ref[...], s_ref[pl.ds(c * _BK, _BK), :],
        (((1,), (0,)), ((), ())),
        precision=jax.lax.Precision.DEFAULT,
        preferred_element_type=jnp.float32)

    @pl.when(ph == 0)
    def _pass1():
        part = res[:, :_F1]

        @pl.when(c == 0)
        def _():
            h1p_ref[...] = part

        @pl.when(c != 0)
        def _():
            h1p_ref[...] = h1p_ref[...] + part

        @pl.when(c == _CB - 1)
        def _finalize_stripe():
            h1 = jnp.maximum(h1p_ref[...] + b0_ref[...], 0.0)
            s2_blk = jnp.dot(h1.astype(jnp.bfloat16),
                             w1_ref[...].astype(jnp.bfloat16),
                             preferred_element_type=jnp.float32)
            s_ref[pl.ds(r * _BM, _BM), _F1:] = s2_blk

    @pl.when(jnp.logical_or(ph == 1, _BK * (c + 1) <= _BM * r))
    def _pass2():
        contrib = res[:, _F1:]

        @pl.when(c == 0)
        def _():
            out_ref[pl.ds(r * _BM, _BM), :] = contrib

        @pl.when(c != 0)
        def _():
            out_ref[pl.ds(r * _BM, _BM), :] = (
                out_ref[pl.ds(r * _BM, _BM), :] + contrib)


def kernel(x, adj, W0, b0, W1, b1):
    x2d = x.reshape(_N, x.shape[-1])
    x_pad = jnp.pad(x2d, ((0, _NPAD - _N), (0, 0)))
    s1 = pl.pallas_call(
        _mm_kernel,
        out_shape=jax.ShapeDtypeStruct((_NPAD, _F1), jnp.float32),
    )(x_pad, W0)

    h2 = pl.pallas_call(
        _fused_kernel,
        grid_spec=pltpu.PrefetchScalarGridSpec(
            num_scalar_prefetch=3,
            grid=(_NSTEPS,),
            in_specs=[
                pl.BlockSpec((_BM, _BK),
                             lambda t, rt, ct, pt: (rt[t], ct[t])),
                pl.BlockSpec((_NPAD, _F1), lambda t, rt, ct, pt: (0, 0)),
                pl.BlockSpec((1, _F1), lambda t, rt, ct, pt: (0, 0)),
                pl.BlockSpec((_F1, _F2), lambda t, rt, ct, pt: (0, 0)),
            ],
            out_specs=pl.BlockSpec((_N, _F2), lambda t, rt, ct, pt: (0, 0)),
            scratch_shapes=[
                pltpu.VMEM((_BM, _F1), jnp.float32),
                pltpu.VMEM((_NPAD, _F1 + _F2), jnp.float32),
            ],
        ),
        out_shape=jax.ShapeDtypeStruct((_N, _F2), jnp.float32),
        compiler_params=pltpu.CompilerParams(
            dimension_semantics=("arbitrary",)),
    )(jnp.asarray(_RTAB), jnp.asarray(_CTAB), jnp.asarray(_PTAB),
      adj, s1, b0.reshape(1, -1), W1)

    out = pl.pallas_call(
        _lsm_kernel,
        out_shape=jax.ShapeDtypeStruct((_N, _F2), jnp.float32),
    )(h2)
    return out.reshape(1, _N, _F2)


# speedup vs baseline: 1.3446x; 1.0614x over previous
import numpy as np

import jax
import jax.numpy as jnp
from jax.experimental import pallas as pl
from jax.experimental.pallas import tpu as pltpu

_N = 10000
_BM = 1000
_BK = 2560
_RB = _N // _BM
_CB = -(-_N // _BK)
_NPAD = _CB * _BK
_F1 = 128
_F2 = 64


def _dual(r, c):
    return _BK * (c + 1) <= _BM * r


def _make_schedule():
    rs, cs, ph = [], [], []
    for r in range(_RB):
        for c in range(_CB):
            rs.append(r)
            cs.append(c)
            ph.append(0)
    for r in range(_RB):
        for c in range(_CB):
            if not _dual(r, c):
                rs.append(r)
                cs.append(c)
                ph.append(1)
    return (np.asarray(rs, np.int32), np.asarray(cs, np.int32),
            np.asarray(ph, np.int32))


_RTAB, _CTAB, _PTAB = _make_schedule()
_NSTEPS = _RTAB.shape[0]


def _mm_kernel(a_ref, w_ref, o_ref):
    a = a_ref[...].astype(jnp.bfloat16)
    w = w_ref[...].astype(jnp.bfloat16)
    o_ref[...] = jnp.dot(a, w, preferred_element_type=jnp.float32)


def _lsm_kernel(h_ref, o_ref):
    h = h_ref[...]
    m = jnp.max(h, axis=0, keepdims=True)
    lse = jnp.log(jnp.sum(jnp.exp(h - m), axis=0, keepdims=True)) + m
    o_ref[...] = h - lse


def _fused_kernel(rtab_ref, ctab_ref, ptab_ref, adj_ref, x_ref, b0_ref,
                  w0_ref, w1_ref, out_ref, h1p_ref, s_ref):
    t = pl.program_id(0)
    r = rtab_ref[t]
    c = ctab_ref[t]
    ph = ptab_ref[t]

    @pl.when(t == 0)
    def _init_s():
        s1 = jnp.dot(x_ref[...].astype(jnp.bfloat16),
                     w0_ref[...].astype(jnp.bfloat16),
                     preferred_element_type=jnp.float32)
        s_ref[pl.ds(0, _N), :_F1] = s1
        s_ref[pl.ds(_N, _NPAD - _N), :_F1] = jnp.zeros(
            (_NPAD - _N, _F1), jnp.float32)
        s_ref[:, _F1:] = jnp.zeros((_NPAD, _F2), jnp.float32)

    res = jax.lax.dot_general(
        adj_ref[...], s_ref[pl.ds(c * _BK, _BK), :],
        (((1,), (0,)), ((), ())),
        precision=jax.lax.Precision.DEFAULT,
        preferred_element_type=jnp.float32)

    @pl.when(ph == 0)
    def _pass1():
        part = res[:, :_F1]

        @pl.when(c == 0)
        def _():
            h1p_ref[...] = part

        @pl.when(c != 0)
        def _():
            h1p_ref[...] = h1p_ref[...] + part

        @pl.when(c == _CB - 1)
        def _finalize_stripe():
            h1 = jnp.maximum(h1p_ref[...] + b0_ref[...], 0.0)
            s2_blk = jnp.dot(h1.astype(jnp.bfloat16),
                             w1_ref[...].astype(jnp.bfloat16),
                             preferred_element_type=jnp.float32)
            s_ref[pl.ds(r * _BM, _BM), _F1:] = s2_blk

    @pl.when(jnp.logical_or(ph == 1, _BK * (c + 1) <= _BM * r))
    def _pass2():
        contrib = res[:, _F1:]

        @pl.when(c == 0)
        def _():
            out_ref[pl.ds(r * _BM, _BM), :] = contrib

        @pl.when(c != 0)
        def _():
            out_ref[pl.ds(r * _BM, _BM), :] = (
                out_ref[pl.ds(r * _BM, _BM), :] + contrib)

    @pl.when(t == _NSTEPS - 1)
    def _log_softmax():
        h = out_ref[...]
        m = jnp.max(h, axis=0, keepdims=True)
        lse = jnp.log(jnp.sum(jnp.exp(h - m), axis=0, keepdims=True)) + m
        out_ref[...] = h - lse


def kernel(x, adj, W0, b0, W1, b1):
    x2d = x.reshape(_N, x.shape[-1])
    out = pl.pallas_call(
        _fused_kernel,
        grid_spec=pltpu.PrefetchScalarGridSpec(
            num_scalar_prefetch=3,
            grid=(_NSTEPS,),
            in_specs=[
                pl.BlockSpec((_BM, _BK),
                             lambda t, rt, ct, pt: (rt[t], ct[t])),
                pl.BlockSpec((_N, _F1), lambda t, rt, ct, pt: (0, 0)),
                pl.BlockSpec((1, _F1), lambda t, rt, ct, pt: (0, 0)),
                pl.BlockSpec((_F1, _F1), lambda t, rt, ct, pt: (0, 0)),
                pl.BlockSpec((_F1, _F2), lambda t, rt, ct, pt: (0, 0)),
            ],
            out_specs=pl.BlockSpec((_N, _F2), lambda t, rt, ct, pt: (0, 0)),
            scratch_shapes=[
                pltpu.VMEM((_BM, _F1), jnp.float32),
                pltpu.VMEM((_NPAD, _F1 + _F2), jnp.float32),
            ],
        ),
        out_shape=jax.ShapeDtypeStruct((_N, _F2), jnp.float32),
        compiler_params=pltpu.CompilerParams(
            dimension_semantics=("arbitrary",)),
    )(jnp.asarray(_RTAB), jnp.asarray(_CTAB), jnp.asarray(_PTAB),
      adj, x2d, b0.reshape(1, -1), W0, W1)
    return out.reshape(1, _N, _F2)


# speedup vs baseline: 1.3569x; 1.0091x over previous
import numpy as np

import jax
import jax.numpy as jnp
from jax.experimental import pallas as pl
from jax.experimental.pallas import tpu as pltpu

_N = 10000
_BM = 1000
_BK = 2560
_RB = _N // _BM
_CB = -(-_N // _BK)
_NPAD = _CB * _BK
_F1 = 128
_F2 = 64


def _dual(r, c):
    return (_BK * (c + 1) <= _BM * r) or (c == _CB - 1 and r == _RB - 1)


def _make_schedule():
    rs, cs, ph = [], [], []
    for r in range(_RB):
        for c in range(_CB):
            rs.append(r)
            cs.append(c)
            ph.append(0)
    for r in range(_RB):
        for c in range(_CB):
            if not _dual(r, c):
                rs.append(r)
                cs.append(c)
                ph.append(1)
    return (np.asarray(rs, np.int32), np.asarray(cs, np.int32),
            np.asarray(ph, np.int32))


_RTAB, _CTAB, _PTAB = _make_schedule()
_NSTEPS = _RTAB.shape[0]


def _mm_kernel(a_ref, w_ref, o_ref):
    a = a_ref[...].astype(jnp.bfloat16)
    w = w_ref[...].astype(jnp.bfloat16)
    o_ref[...] = jnp.dot(a, w, preferred_element_type=jnp.float32)


def _lsm_kernel(h_ref, o_ref):
    h = h_ref[...]
    m = jnp.max(h, axis=0, keepdims=True)
    lse = jnp.log(jnp.sum(jnp.exp(h - m), axis=0, keepdims=True)) + m
    o_ref[...] = h - lse


def _fused_kernel(rtab_ref, ctab_ref, ptab_ref, adj_ref, x_ref, b0_ref,
                  w0_ref, w1_ref, out_ref, h1p_ref, s_ref):
    t = pl.program_id(0)
    r = rtab_ref[t]
    c = ctab_ref[t]
    ph = ptab_ref[t]

    @pl.when(t == 0)
    def _init_s():
        s1 = jnp.dot(x_ref[...].astype(jnp.bfloat16),
                     w0_ref[...].astype(jnp.bfloat16),
                     preferred_element_type=jnp.float32)
        s_ref[pl.ds(0, _N), :_F1] = s1
        s_ref[pl.ds(_N, _NPAD - _N), :_F1] = jnp.zeros(
            (_NPAD - _N, _F1), jnp.float32)
        s_ref[:, _F1:] = jnp.zeros((_NPAD, _F2), jnp.float32)

    res = jax.lax.dot_general(
        adj_ref[...], s_ref[pl.ds(c * _BK, _BK), :],
        (((1,), (0,)), ((), ())),
        precision=jax.lax.Precision.DEFAULT,
        preferred_element_type=jnp.float32)

    @pl.when(ph == 0)
    def _pass1():
        part = res[:, :_F1]

        @pl.when(c == 0)
        def _():
            h1p_ref[...] = part

        @pl.when(c != 0)
        def _():
            h1p_ref[...] = h1p_ref[...] + part

        @pl.when(c == _CB - 1)
        def _finalize_stripe():
            h1 = jnp.maximum(h1p_ref[...] + b0_ref[...], 0.0)
            s2_blk = jnp.dot(h1.astype(jnp.bfloat16),
                             w1_ref[...].astype(jnp.bfloat16),
                             preferred_element_type=jnp.float32)
            s_ref[pl.ds(r * _BM, _BM), _F1:] = s2_blk

    is_dual = jnp.logical_or(
        _BK * (c + 1) <= _BM * r,
        jnp.logical_and(c == _CB - 1, r == _RB - 1))

    @pl.when(jnp.logical_or(ph == 1, is_dual))
    def _pass2():
        contrib = res[:, _F1:]

        @pl.when(c == 0)
        def _():
            out_ref[pl.ds(r * _BM, _BM), :] = contrib

        @pl.when(c != 0)
        def _():
            out_ref[pl.ds(r * _BM, _BM), :] = (
                out_ref[pl.ds(r * _BM, _BM), :] + contrib)

    @pl.when(t == _NSTEPS - 1)
    def _log_softmax():
        h = out_ref[...]
        m = jnp.max(h, axis=0, keepdims=True)
        lse = jnp.log(jnp.sum(jnp.exp(h - m), axis=0, keepdims=True)) + m
        out_ref[...] = h - lse


def kernel(x, adj, W0, b0, W1, b1):
    x2d = x.reshape(_N, x.shape[-1])
    out = pl.pallas_call(
        _fused_kernel,
        grid_spec=pltpu.PrefetchScalarGridSpec(
            num_scalar_prefetch=3,
            grid=(_NSTEPS,),
            in_specs=[
                pl.BlockSpec((_BM, _BK),
                             lambda t, rt, ct, pt: (rt[t], ct[t])),
                pl.BlockSpec((_N, _F1), lambda t, rt, ct, pt: (0, 0)),
                pl.BlockSpec((1, _F1), lambda t, rt, ct, pt: (0, 0)),
                pl.BlockSpec((_F1, _F1), lambda t, rt, ct, pt: (0, 0)),
                pl.BlockSpec((_F1, _F2), lambda t, rt, ct, pt: (0, 0)),
            ],
            out_specs=pl.BlockSpec((_N, _F2), lambda t, rt, ct, pt: (0, 0)),
            scratch_shapes=[
                pltpu.VMEM((_BM, _F1), jnp.float32),
                pltpu.VMEM((_NPAD, _F1 + _F2), jnp.float32),
            ],
        ),
        out_shape=jax.ShapeDtypeStruct((_N, _F2), jnp.float32),
        compiler_params=pltpu.CompilerParams(
            dimension_semantics=("arbitrary",)),
    )(jnp.asarray(_RTAB), jnp.asarray(_CTAB), jnp.asarray(_PTAB),
      adj, x2d, b0.reshape(1, -1), W0, W1)
    return out.reshape(1, _N, _F2)
